# baseline (speedup 1.0000x reference)
import functools

import jax
import jax.numpy as jnp
from jax import lax
from jax.experimental import pallas as pl
from jax.experimental.pallas import tpu as pltpu
from jax.experimental.pallas import tpu_sc as plsc

N = 10000
E = 320000
NHID = 128
NLAYER = 3
NGRAPH = 128
NCLASS = 10

CHUNK = 128
NCORES = 2
NSUB = 16
NWORKERS = NCORES * NSUB
CHLOC = 80
NCHUNKS = NWORKERS * CHLOC
E_PAD = NCHUNKS * CHUNK
NBUF = 2
CHHALF = CHLOC // 2
N_PAD = N + 8
ROWS_A = 624
ROWS_LAST = N_PAD - (NSUB - 1) * ROWS_A

BR = 1000
NBLK = N // BR


def _sc_aggregate(h_pad, src2, dst2):
    mesh = plsc.VectorSubcoreMesh(core_axis_name="c", subcore_axis_name="s")

    @functools.partial(
        pl.kernel,
        mesh=mesh,
        out_type=jax.ShapeDtypeStruct((NCORES, N_PAD, NHID), jnp.float32),
        scratch_types=[
            pltpu.VMEM((CHHALF, CHUNK), jnp.int32),
            pltpu.VMEM((CHHALF, CHUNK), jnp.int32),
        ] + [pltpu.VMEM((CHUNK, NHID), jnp.float32) for _ in range(NBUF)] + [
            pltpu.VMEM_SHARED((N_PAD, NHID), jnp.float32),
        ] + [pltpu.SemaphoreType.DMA for _ in range(NBUF)],
    )
    def agg_kernel(h_hbm, src_hbm, dst_hbm, out_hbm, src_buf, dst_buf,
                   *scratches):
        rows = scratches[:NBUF]
        acc = scratches[NBUF]
        gsem = scratches[NBUF + 1:2 * NBUF + 1]

        c = lax.axis_index("c")
        s = lax.axis_index("s")
        w = s * NCORES + c

        row0 = pl.multiple_of(s * ROWS_A, 8)

        @pl.when(s < NSUB - 1)
        def _init_a():
            pltpu.sync_copy(h_hbm.at[pl.ds(row0, ROWS_A)],
                            acc.at[pl.ds(row0, ROWS_A)])

        @pl.when(s == NSUB - 1)
        def _init_b():
            pltpu.sync_copy(h_hbm.at[pl.ds((NSUB - 1) * ROWS_A, ROWS_LAST)],
                            acc.at[pl.ds((NSUB - 1) * ROWS_A, ROWS_LAST)])

        plsc.subcore_barrier()

        def g_start(j, b):
            pltpu.async_copy(h_hbm.at[src_buf.at[j]], rows[b], gsem[b])

        def g_wait(b):
            pltpu.make_async_copy(h_hbm.at[src_buf.at[0]], rows[b],
                                  gsem[b]).wait()

        def s_add(j, b):
            pltpu.sync_copy(rows[b], acc.at[dst_buf.at[j]], add=True)

        for half in range(2):
            ch0 = pl.multiple_of(w * CHLOC + half * CHHALF, 8)
            pltpu.sync_copy(src_hbm.at[pl.ds(ch0, CHHALF)], src_buf)
            pltpu.sync_copy(dst_hbm.at[pl.ds(ch0, CHHALF)], dst_buf)

            for b in range(NBUF):
                g_start(b, b)

            @pl.loop(0, CHHALF - NBUF, step=NBUF)
            def _ring(g):
                for b in range(NBUF):
                    j = g + b
                    g_wait(b)
                    s_add(j, b)
                    g_start(j + NBUF, b)

            for b in range(NBUF):
                j = CHHALF - NBUF + b
                g_wait(b)
                s_add(j, b)

        plsc.subcore_barrier()

        @pl.when(s < NSUB - 1)
        def _out_a():
            pltpu.sync_copy(acc.at[pl.ds(row0, ROWS_A)],
                            out_hbm.at[c, pl.ds(row0, ROWS_A)])

        @pl.when(s == NSUB - 1)
        def _out_b():
            pltpu.sync_copy(acc.at[pl.ds((NSUB - 1) * ROWS_A, ROWS_LAST)],
                            out_hbm.at[c, pl.ds((NSUB - 1) * ROWS_A, ROWS_LAST)])

    return agg_kernel(h_pad, src2, dst2)


def _mlp_body(p_ref, h_ref, w1_ref, b1_ref, w2_ref, b2_ref, o_ref):
    z = p_ref[0] + p_ref[1] - h_ref[...]
    z = jnp.dot(z, w1_ref[...], preferred_element_type=jnp.float32) + b1_ref[...]
    z = jnp.maximum(z, 0.0)
    z = jnp.dot(z, w2_ref[...], preferred_element_type=jnp.float32) + b2_ref[...]
    o_ref[...] = jnp.maximum(z, 0.0)


def _tc_mlp(p, h, w1, b1, w2, b2):
    return pl.pallas_call(
        _mlp_body,
        grid=(NBLK,),
        in_specs=[
            pl.BlockSpec((NCORES, BR, NHID), lambda i: (0, i, 0)),
            pl.BlockSpec((BR, NHID), lambda i: (i, 0)),
            pl.BlockSpec((NHID, NHID), lambda i: (0, 0)),
            pl.BlockSpec((1, NHID), lambda i: (0, 0)),
            pl.BlockSpec((NHID, NHID), lambda i: (0, 0)),
            pl.BlockSpec((1, NHID), lambda i: (0, 0)),
        ],
        out_specs=pl.BlockSpec((BR, NHID), lambda i: (i, 0)),
        out_shape=jax.ShapeDtypeStruct((N, NHID), jnp.float32),
    )(p, h, w1, b1, w2, b2)


def _pool_head_body(h_ref, b_ref, pw_ref, pb_ref, rw_ref, rb_ref, o_ref, g_acc):
    i = pl.program_id(0)

    @pl.when(i == 0)
    def _init():
        g_acc[...] = jnp.zeros_like(g_acc)

    bvals = b_ref[0]
    mask = (lax.broadcasted_iota(jnp.int32, (NGRAPH, BR), 0) == bvals
            ).astype(jnp.float32)
    g_acc[...] += jnp.dot(mask, h_ref[...], preferred_element_type=jnp.float32)

    @pl.when(i == NBLK - 1)
    def _head():
        g = g_acc[...]
        z = jnp.dot(g, pw_ref[...], preferred_element_type=jnp.float32) + pb_ref[...]
        z = jnp.maximum(z, 0.0)
        logits = jnp.dot(z, rw_ref[...], preferred_element_type=jnp.float32) + rb_ref[...]
        col = lax.broadcasted_iota(jnp.int32, (NGRAPH, NHID), 1)
        logits = jnp.where(col < NCLASS, logits, jnp.float32(-1e30))
        m = jnp.max(logits, axis=1, keepdims=True)
        lse = m + jnp.log(jnp.sum(jnp.exp(logits - m), axis=1, keepdims=True))
        o_ref[...] = logits - lse


def _tc_pool_head(h, batch3, post_w, post_b, rw_pad, rb_pad):
    return pl.pallas_call(
        _pool_head_body,
        grid=(NBLK,),
        in_specs=[
            pl.BlockSpec((BR, NHID), lambda i: (i, 0)),
            pl.BlockSpec((1, 1, BR), lambda i: (i, 0, 0)),
            pl.BlockSpec((NHID, NHID), lambda i: (0, 0)),
            pl.BlockSpec((1, NHID), lambda i: (0, 0)),
            pl.BlockSpec((NHID, NHID), lambda i: (0, 0)),
            pl.BlockSpec((1, NHID), lambda i: (0, 0)),
        ],
        out_specs=pl.BlockSpec((NGRAPH, NHID), lambda i: (0, 0)),
        out_shape=jax.ShapeDtypeStruct((NGRAPH, NHID), jnp.float32),
        scratch_shapes=[pltpu.VMEM((NGRAPH, NHID), jnp.float32)],
    )(h, batch3, post_w, post_b, rw_pad, rb_pad)


def kernel(x, edge_index, batch, conv_w1, conv_b1, conv_w2, conv_b2,
           post_w, post_b, read_w, read_b):
    pad = jnp.full((E_PAD - E,), N, jnp.int32)
    src2 = jnp.concatenate([edge_index[0], pad]).reshape(NCHUNKS, CHUNK)
    dst2 = jnp.concatenate([edge_index[1], pad]).reshape(NCHUNKS, CHUNK)
    h = x
    for l in range(NLAYER):
        h_pad = jnp.pad(h, ((0, N_PAD - N), (0, 0)))
        p = _sc_aggregate(h_pad, src2, dst2)
        h = _tc_mlp(p, h, conv_w1[l], conv_b1[l].reshape(1, NHID),
                    conv_w2[l], conv_b2[l].reshape(1, NHID))

    batch3 = batch.reshape(NBLK, 1, BR)
    rw_pad = jnp.zeros((NHID, NHID), jnp.float32).at[:, :NCLASS].set(read_w)
    rb_pad = jnp.zeros((1, NHID), jnp.float32).at[0, :NCLASS].set(read_b)
    out = _tc_pool_head(h, batch3, post_w, post_b.reshape(1, NHID),
                        rw_pad, rb_pad)
    return out[:, :NCLASS]


# speedup vs baseline: 1.1041x; 1.1041x over previous
import functools

import jax
import jax.numpy as jnp
from jax import lax
from jax.experimental import pallas as pl
from jax.experimental.pallas import tpu as pltpu
from jax.experimental.pallas import tpu_sc as plsc

N = 10000
E = 320000
NHID = 128
NLAYER = 3
NGRAPH = 128
NCLASS = 10

CHUNK = 128
NCORES = 2
NSUB = 16
NWORKERS = NCORES * NSUB
CHLOC = 80
NCHUNKS = NWORKERS * CHLOC
E_PAD = NCHUNKS * CHUNK
NBUF = 2
CHHALF = CHLOC // 2
N_PAD = N + 8
ROWS_A = 624
ROWS_LAST = N_PAD - (NSUB - 1) * ROWS_A

BR = 1000
NBLK = N // BR


def _sc_aggregate(h_pad, src2, dst2):
    mesh = plsc.VectorSubcoreMesh(core_axis_name="c", subcore_axis_name="s")

    @functools.partial(
        pl.kernel,
        mesh=mesh,
        out_type=jax.ShapeDtypeStruct((NCORES, N_PAD, NHID), jnp.float32),
        scratch_types=[
            pltpu.VMEM((CHHALF, CHUNK), jnp.int32),
            pltpu.VMEM((CHHALF, CHUNK), jnp.int32),
        ] + [pltpu.VMEM((CHUNK, NHID), jnp.float32) for _ in range(NBUF)] + [
            pltpu.VMEM_SHARED((N_PAD, NHID), jnp.float32),
        ] + [pltpu.SemaphoreType.DMA for _ in range(NBUF)],
    )
    def agg_kernel(h_hbm, src_hbm, dst_hbm, out_hbm, src_buf, dst_buf,
                   *scratches):
        rows = scratches[:NBUF]
        acc = scratches[NBUF]
        gsem = scratches[NBUF + 1:2 * NBUF + 1]

        c = lax.axis_index("c")
        s = lax.axis_index("s")
        w = s * NCORES + c

        row0 = pl.multiple_of(s * ROWS_A, 8)

        @pl.when(s < NSUB - 1)
        def _init_a():
            pltpu.sync_copy(h_hbm.at[pl.ds(row0, ROWS_A)],
                            acc.at[pl.ds(row0, ROWS_A)])

        @pl.when(s == NSUB - 1)
        def _init_b():
            pltpu.sync_copy(h_hbm.at[pl.ds((NSUB - 1) * ROWS_A, ROWS_LAST)],
                            acc.at[pl.ds((NSUB - 1) * ROWS_A, ROWS_LAST)])

        plsc.subcore_barrier()

        def g_start(j, b):
            pltpu.async_copy(h_hbm.at[src_buf.at[j]], rows[b], gsem[b])

        def g_wait(b):
            pltpu.make_async_copy(h_hbm.at[src_buf.at[0]], rows[b],
                                  gsem[b]).wait()

        def s_add(j, b):
            pltpu.sync_copy(rows[b], acc.at[dst_buf.at[j]], add=True)

        for half in range(2):
            ch0 = pl.multiple_of(w * CHLOC + half * CHHALF, 8)
            pltpu.sync_copy(src_hbm.at[pl.ds(ch0, CHHALF)], src_buf)
            pltpu.sync_copy(dst_hbm.at[pl.ds(ch0, CHHALF)], dst_buf)

            for b in range(NBUF):
                g_start(b, b)

            @pl.loop(0, CHHALF - NBUF, step=NBUF)
            def _ring(g):
                for b in range(NBUF):
                    j = g + b
                    g_wait(b)
                    s_add(j, b)
                    g_start(j + NBUF, b)

            for b in range(NBUF):
                j = CHHALF - NBUF + b
                g_wait(b)
                s_add(j, b)

        plsc.subcore_barrier()

        @pl.when(s < NSUB - 1)
        def _out_a():
            pltpu.sync_copy(acc.at[pl.ds(row0, ROWS_A)],
                            out_hbm.at[c, pl.ds(row0, ROWS_A)])

        @pl.when(s == NSUB - 1)
        def _out_b():
            pltpu.sync_copy(acc.at[pl.ds((NSUB - 1) * ROWS_A, ROWS_LAST)],
                            out_hbm.at[c, pl.ds((NSUB - 1) * ROWS_A, ROWS_LAST)])

    return agg_kernel(h_pad, src2, dst2)


def _mlp_body(p_ref, h_ref, w1_ref, b1_ref, w2_ref, b2_ref, o_ref):
    z = p_ref[0] + p_ref[1] - h_ref[...]
    z = jnp.dot(z, w1_ref[...], preferred_element_type=jnp.float32) + b1_ref[...]
    z = jnp.maximum(z, 0.0)
    z = jnp.dot(z, w2_ref[...], preferred_element_type=jnp.float32) + b2_ref[...]
    o_ref[...] = jnp.maximum(z, 0.0)


def _tc_mlp(p, h, w1, b1, w2, b2):
    return pl.pallas_call(
        _mlp_body,
        grid=(NBLK,),
        in_specs=[
            pl.BlockSpec((NCORES, BR, NHID), lambda i: (0, i, 0)),
            pl.BlockSpec((BR, NHID), lambda i: (i, 0)),
            pl.BlockSpec((NHID, NHID), lambda i: (0, 0)),
            pl.BlockSpec((1, NHID), lambda i: (0, 0)),
            pl.BlockSpec((NHID, NHID), lambda i: (0, 0)),
            pl.BlockSpec((1, NHID), lambda i: (0, 0)),
        ],
        out_specs=pl.BlockSpec((BR, NHID), lambda i: (i, 0)),
        out_shape=jax.ShapeDtypeStruct((N, NHID), jnp.float32),
    )(p, h, w1, b1, w2, b2)


def _pool_head_body(h_ref, b_ref, pw_ref, pb_ref, rw_ref, rb_ref, o_ref, g_acc):
    i = pl.program_id(0)

    @pl.when(i == 0)
    def _init():
        g_acc[...] = jnp.zeros_like(g_acc)

    bvals = b_ref[0]
    mask = (lax.broadcasted_iota(jnp.int32, (NGRAPH, BR), 0) == bvals
            ).astype(jnp.float32)
    g_acc[...] += jnp.dot(mask, h_ref[...], preferred_element_type=jnp.float32)

    @pl.when(i == NBLK - 1)
    def _head():
        g = g_acc[...]
        z = jnp.dot(g, pw_ref[...], preferred_element_type=jnp.float32) + pb_ref[...]
        z = jnp.maximum(z, 0.0)
        logits = jnp.dot(z, rw_ref[...], preferred_element_type=jnp.float32) + rb_ref[...]
        col = lax.broadcasted_iota(jnp.int32, (NGRAPH, NHID), 1)
        logits = jnp.where(col < NCLASS, logits, jnp.float32(-1e30))
        m = jnp.max(logits, axis=1, keepdims=True)
        lse = m + jnp.log(jnp.sum(jnp.exp(logits - m), axis=1, keepdims=True))
        o_ref[...] = logits - lse


def _tc_pool_head(h, batch3, post_w, post_b, rw_pad, rb_pad):
    return pl.pallas_call(
        _pool_head_body,
        grid=(NBLK,),
        in_specs=[
            pl.BlockSpec((BR, NHID), lambda i: (i, 0)),
            pl.BlockSpec((1, 1, BR), lambda i: (i, 0, 0)),
            pl.BlockSpec((NHID, NHID), lambda i: (0, 0)),
            pl.BlockSpec((1, NHID), lambda i: (0, 0)),
            pl.BlockSpec((NHID, NHID), lambda i: (0, 0)),
            pl.BlockSpec((1, NHID), lambda i: (0, 0)),
        ],
        out_specs=pl.BlockSpec((NGRAPH, NHID), lambda i: (0, 0)),
        out_shape=jax.ShapeDtypeStruct((NGRAPH, NHID), jnp.float32),
        scratch_shapes=[pltpu.VMEM((NGRAPH, NHID), jnp.float32)],
    )(h, batch3, post_w, post_b, rw_pad, rb_pad)


def kernel(x, edge_index, batch, conv_w1, conv_b1, conv_w2, conv_b2,
           post_w, post_b, read_w, read_b):
    pad_src = jnp.full((E_PAD - E,), N, jnp.int32)
    pad_dst = jnp.arange(E_PAD - E, dtype=jnp.int32) % N
    src2 = jnp.concatenate([edge_index[0], pad_src]).reshape(NCHUNKS, CHUNK)
    dst2 = jnp.concatenate([edge_index[1], pad_dst]).reshape(NCHUNKS, CHUNK)
    h = x
    for l in range(NLAYER):
        h_pad = jnp.pad(h, ((0, N_PAD - N), (0, 0)))
        p = _sc_aggregate(h_pad, src2, dst2)
        h = _tc_mlp(p, h, conv_w1[l], conv_b1[l].reshape(1, NHID),
                    conv_w2[l], conv_b2[l].reshape(1, NHID))

    batch3 = batch.reshape(NBLK, 1, BR)
    rw_pad = jnp.zeros((NHID, NHID), jnp.float32).at[:, :NCLASS].set(read_w)
    rb_pad = jnp.zeros((1, NHID), jnp.float32).at[0, :NCLASS].set(read_b)
    out = _tc_pool_head(h, batch3, post_w, post_b.reshape(1, NHID),
                        rw_pad, rb_pad)
    return out[:, :NCLASS]


# speedup vs baseline: 3.5488x; 3.2142x over previous
import functools

import jax
import jax.numpy as jnp
from jax import lax
from jax.experimental import pallas as pl
from jax.experimental.pallas import tpu as pltpu
from jax.experimental.pallas import tpu_sc as plsc

N = 10000
E = 320000
NHID = 128
NLAYER = 3
NGRAPH = 128
NCLASS = 10

CHUNK = 128
NCORES = 2
NSUB = 16
NWORKERS = NCORES * NSUB
CHLOC = 80
NCHUNKS = NWORKERS * CHLOC
E_PAD = NCHUNKS * CHUNK
NBUF = 2
CHHALF = CHLOC // 2
EDGES_W = E // NWORKERS
PADS_W = CHLOC * CHUNK - EDGES_W
N_PAD = N + PADS_W
ROWS_T = N_PAD // NSUB

BR = 1000
NBLK = N // BR


def _sc_aggregate(h_pad, src2, dst2):
    mesh = plsc.VectorSubcoreMesh(core_axis_name="c", subcore_axis_name="s")

    @functools.partial(
        pl.kernel,
        mesh=mesh,
        out_type=jax.ShapeDtypeStruct((NCORES, N_PAD, NHID), jnp.float32),
        scratch_types=[
            pltpu.VMEM((CHHALF, CHUNK), jnp.int32),
            pltpu.VMEM((CHHALF, CHUNK), jnp.int32),
        ] + [pltpu.VMEM((CHUNK, NHID), jnp.float32) for _ in range(NBUF)] + [
            pltpu.VMEM_SHARED((N_PAD, NHID), jnp.float32),
        ] + [pltpu.SemaphoreType.DMA for _ in range(NBUF)],
    )
    def agg_kernel(h_hbm, src_hbm, dst_hbm, out_hbm, src_buf, dst_buf,
                   *scratches):
        rows = scratches[:NBUF]
        acc = scratches[NBUF]
        gsem = scratches[NBUF + 1:2 * NBUF + 1]

        c = lax.axis_index("c")
        s = lax.axis_index("s")
        w = s * NCORES + c

        row0 = pl.multiple_of(s * ROWS_T, 8)
        pltpu.sync_copy(h_hbm.at[pl.ds(row0, ROWS_T)],
                        acc.at[pl.ds(row0, ROWS_T)])
        plsc.subcore_barrier()

        def g_start(j, b):
            pltpu.async_copy(h_hbm.at[src_buf.at[j]], rows[b], gsem[b])

        def g_wait(b):
            pltpu.make_async_copy(h_hbm.at[src_buf.at[0]], rows[b],
                                  gsem[b]).wait()

        def s_add(j, b):
            pltpu.sync_copy(rows[b], acc.at[dst_buf.at[j]], add=True)

        for half in range(2):
            ch0 = pl.multiple_of(w * CHLOC + half * CHHALF, 8)
            pltpu.sync_copy(src_hbm.at[pl.ds(ch0, CHHALF)], src_buf)
            pltpu.sync_copy(dst_hbm.at[pl.ds(ch0, CHHALF)], dst_buf)

            for b in range(NBUF):
                g_start(b, b)

            @pl.loop(0, CHHALF - NBUF, step=NBUF)
            def _ring(g):
                for b in range(NBUF):
                    j = g + b
                    g_wait(b)
                    s_add(j, b)
                    g_start(j + NBUF, b)

            for b in range(NBUF):
                j = CHHALF - NBUF + b
                g_wait(b)
                s_add(j, b)

        plsc.subcore_barrier()

        pltpu.sync_copy(acc.at[pl.ds(row0, ROWS_T)],
                        out_hbm.at[c, pl.ds(row0, ROWS_T)])

    return agg_kernel(h_pad, src2, dst2)


def _mlp_body(p_ref, h_ref, w1_ref, b1_ref, w2_ref, b2_ref, o_ref):
    z = p_ref[0] + p_ref[1] - h_ref[...]
    z = jnp.dot(z, w1_ref[...], preferred_element_type=jnp.float32) + b1_ref[...]
    z = jnp.maximum(z, 0.0)
    z = jnp.dot(z, w2_ref[...], preferred_element_type=jnp.float32) + b2_ref[...]
    o_ref[...] = jnp.maximum(z, 0.0)


def _tc_mlp(p, h, w1, b1, w2, b2):
    return pl.pallas_call(
        _mlp_body,
        grid=(NBLK,),
        in_specs=[
            pl.BlockSpec((NCORES, BR, NHID), lambda i: (0, i, 0)),
            pl.BlockSpec((BR, NHID), lambda i: (i, 0)),
            pl.BlockSpec((NHID, NHID), lambda i: (0, 0)),
            pl.BlockSpec((1, NHID), lambda i: (0, 0)),
            pl.BlockSpec((NHID, NHID), lambda i: (0, 0)),
            pl.BlockSpec((1, NHID), lambda i: (0, 0)),
        ],
        out_specs=pl.BlockSpec((BR, NHID), lambda i: (i, 0)),
        out_shape=jax.ShapeDtypeStruct((N, NHID), jnp.float32),
    )(p, h, w1, b1, w2, b2)


def _pool_head_body(h_ref, b_ref, pw_ref, pb_ref, rw_ref, rb_ref, o_ref, g_acc):
    i = pl.program_id(0)

    @pl.when(i == 0)
    def _init():
        g_acc[...] = jnp.zeros_like(g_acc)

    bvals = b_ref[0]
    mask = (lax.broadcasted_iota(jnp.int32, (NGRAPH, BR), 0) == bvals
            ).astype(jnp.float32)
    g_acc[...] += jnp.dot(mask, h_ref[...], preferred_element_type=jnp.float32)

    @pl.when(i == NBLK - 1)
    def _head():
        g = g_acc[...]
        z = jnp.dot(g, pw_ref[...], preferred_element_type=jnp.float32) + pb_ref[...]
        z = jnp.maximum(z, 0.0)
        logits = jnp.dot(z, rw_ref[...], preferred_element_type=jnp.float32) + rb_ref[...]
        col = lax.broadcasted_iota(jnp.int32, (NGRAPH, NHID), 1)
        logits = jnp.where(col < NCLASS, logits, jnp.float32(-1e30))
        m = jnp.max(logits, axis=1, keepdims=True)
        lse = m + jnp.log(jnp.sum(jnp.exp(logits - m), axis=1, keepdims=True))
        o_ref[...] = logits - lse


def _tc_pool_head(h, batch3, post_w, post_b, rw_pad, rb_pad):
    return pl.pallas_call(
        _pool_head_body,
        grid=(NBLK,),
        in_specs=[
            pl.BlockSpec((BR, NHID), lambda i: (i, 0)),
            pl.BlockSpec((1, 1, BR), lambda i: (i, 0, 0)),
            pl.BlockSpec((NHID, NHID), lambda i: (0, 0)),
            pl.BlockSpec((1, NHID), lambda i: (0, 0)),
            pl.BlockSpec((NHID, NHID), lambda i: (0, 0)),
            pl.BlockSpec((1, NHID), lambda i: (0, 0)),
        ],
        out_specs=pl.BlockSpec((NGRAPH, NHID), lambda i: (0, 0)),
        out_shape=jax.ShapeDtypeStruct((NGRAPH, NHID), jnp.float32),
        scratch_shapes=[pltpu.VMEM((NGRAPH, NHID), jnp.float32)],
    )(h, batch3, post_w, post_b, rw_pad, rb_pad)


def kernel(x, edge_index, batch, conv_w1, conv_b1, conv_w2, conv_b2,
           post_w, post_b, read_w, read_b):
    pad_src = jnp.broadcast_to(N + jnp.arange(PADS_W, dtype=jnp.int32),
                               (NWORKERS, PADS_W))
    pad_dst = (jnp.arange(NWORKERS, dtype=jnp.int32)[:, None] * 313
               + jnp.arange(PADS_W, dtype=jnp.int32)[None, :]) % N
    src2 = jnp.concatenate(
        [edge_index[0].reshape(NWORKERS, EDGES_W), pad_src],
        axis=1).reshape(NCHUNKS, CHUNK)
    dst2 = jnp.concatenate(
        [edge_index[1].reshape(NWORKERS, EDGES_W), pad_dst],
        axis=1).reshape(NCHUNKS, CHUNK)
    h = x
    for l in range(NLAYER):
        h_pad = jnp.pad(h, ((0, N_PAD - N), (0, 0)))
        p = _sc_aggregate(h_pad, src2, dst2)
        h = _tc_mlp(p, h, conv_w1[l], conv_b1[l].reshape(1, NHID),
                    conv_w2[l], conv_b2[l].reshape(1, NHID))

    batch3 = batch.reshape(NBLK, 1, BR)
    rw_pad = jnp.zeros((NHID, NHID), jnp.float32).at[:, :NCLASS].set(read_w)
    rb_pad = jnp.zeros((1, NHID), jnp.float32).at[0, :NCLASS].set(read_b)
    out = _tc_pool_head(h, batch3, post_w, post_b.reshape(1, NHID),
                        rw_pad, rb_pad)
    return out[:, :NCLASS]


# speedup vs baseline: 3.7927x; 1.0687x over previous
import functools

import jax
import jax.numpy as jnp
from jax import lax
from jax.experimental import pallas as pl
from jax.experimental.pallas import tpu as pltpu
from jax.experimental.pallas import tpu_sc as plsc

N = 10000
E = 320000
NHID = 128
NLAYER = 3
NGRAPH = 128
NCLASS = 10

CHUNK = 128
NCORES = 2
NSUB = 16
NWORKERS = NCORES * NSUB
CHLOC = 80
NCHUNKS = NWORKERS * CHLOC
E_PAD = NCHUNKS * CHUNK
NBUF = 2
CHHALF = CHLOC // 2
EDGES_W = E // NWORKERS
PADS_W = CHLOC * CHUNK - EDGES_W
N_PAD = N + PADS_W
ROWS_T = N_PAD // NSUB

BR = 1024
NBLK = N_PAD // BR


def _sc_aggregate(h_pad, src2, dst2):
    mesh = plsc.VectorSubcoreMesh(core_axis_name="c", subcore_axis_name="s")

    @functools.partial(
        pl.kernel,
        mesh=mesh,
        out_type=jax.ShapeDtypeStruct((NCORES, N_PAD, NHID), jnp.float32),
        scratch_types=[
            pltpu.VMEM((CHHALF, CHUNK), jnp.int32),
            pltpu.VMEM((CHHALF, CHUNK), jnp.int32),
        ] + [pltpu.VMEM((CHUNK, NHID), jnp.float32) for _ in range(NBUF)] + [
            pltpu.VMEM_SHARED((N_PAD, NHID), jnp.float32),
        ] + [pltpu.SemaphoreType.DMA for _ in range(NBUF)],
    )
    def agg_kernel(h_hbm, src_hbm, dst_hbm, out_hbm, src_buf, dst_buf,
                   *scratches):
        rows = scratches[:NBUF]
        acc = scratches[NBUF]
        gsem = scratches[NBUF + 1:2 * NBUF + 1]

        c = lax.axis_index("c")
        s = lax.axis_index("s")
        w = s * NCORES + c

        row0 = pl.multiple_of(s * ROWS_T, 8)
        pltpu.sync_copy(h_hbm.at[pl.ds(row0, ROWS_T)],
                        acc.at[pl.ds(row0, ROWS_T)])
        plsc.subcore_barrier()

        def g_start(j, b):
            pltpu.async_copy(h_hbm.at[src_buf.at[j]], rows[b], gsem[b])

        def g_wait(b):
            pltpu.make_async_copy(h_hbm.at[src_buf.at[0]], rows[b],
                                  gsem[b]).wait()

        def s_add(j, b):
            pltpu.sync_copy(rows[b], acc.at[dst_buf.at[j]], add=True)

        for half in range(2):
            ch0 = pl.multiple_of(w * CHLOC + half * CHHALF, 8)
            pltpu.sync_copy(src_hbm.at[pl.ds(ch0, CHHALF)], src_buf)
            pltpu.sync_copy(dst_hbm.at[pl.ds(ch0, CHHALF)], dst_buf)

            for b in range(NBUF):
                g_start(b, b)

            @pl.loop(0, CHHALF - NBUF, step=NBUF)
            def _ring(g):
                for b in range(NBUF):
                    j = g + b
                    g_wait(b)
                    s_add(j, b)
                    g_start(j + NBUF, b)

            for b in range(NBUF):
                j = CHHALF - NBUF + b
                g_wait(b)
                s_add(j, b)

        plsc.subcore_barrier()

        pltpu.sync_copy(acc.at[pl.ds(row0, ROWS_T)],
                        out_hbm.at[c, pl.ds(row0, ROWS_T)])

    return agg_kernel(h_pad, src2, dst2)


def _mlp_block(i, p_ref, h_ref, w1_ref, b1_ref, w2_ref, b2_ref):
    z = p_ref[0] + p_ref[1] - h_ref[...]
    z = jnp.dot(z, w1_ref[...], preferred_element_type=jnp.float32) + b1_ref[...]
    z = jnp.maximum(z, 0.0)
    z = jnp.dot(z, w2_ref[...], preferred_element_type=jnp.float32) + b2_ref[...]
    z = jnp.maximum(z, 0.0)
    row = i * BR + lax.broadcasted_iota(jnp.int32, (BR, NHID), 0)
    return jnp.where(row < N, z, 0.0)


def _mlp_body(p_ref, h_ref, w1_ref, b1_ref, w2_ref, b2_ref, o_ref):
    o_ref[...] = _mlp_block(pl.program_id(0), p_ref, h_ref,
                            w1_ref, b1_ref, w2_ref, b2_ref)


_WSPECS = [
    pl.BlockSpec((NHID, NHID), lambda i: (0, 0)),
    pl.BlockSpec((1, NHID), lambda i: (0, 0)),
    pl.BlockSpec((NHID, NHID), lambda i: (0, 0)),
    pl.BlockSpec((1, NHID), lambda i: (0, 0)),
]


def _tc_mlp(p, h, w1, b1, w2, b2):
    return pl.pallas_call(
        _mlp_body,
        grid=(NBLK,),
        in_specs=[
            pl.BlockSpec((NCORES, BR, NHID), lambda i: (0, i, 0)),
            pl.BlockSpec((BR, NHID), lambda i: (i, 0)),
        ] + _WSPECS,
        out_specs=pl.BlockSpec((BR, NHID), lambda i: (i, 0)),
        out_shape=jax.ShapeDtypeStruct((N_PAD, NHID), jnp.float32),
    )(p, h, w1, b1, w2, b2)


def _mlp_pool_head_body(p_ref, h_ref, w1_ref, b1_ref, w2_ref, b2_ref,
                        b_ref, pw_ref, pb_ref, rw_ref, rb_ref, o_ref, g_acc):
    i = pl.program_id(0)

    @pl.when(i == 0)
    def _init():
        g_acc[...] = jnp.zeros_like(g_acc)

    hv = _mlp_block(i, p_ref, h_ref, w1_ref, b1_ref, w2_ref, b2_ref)
    bvals = b_ref[0]
    mask = (lax.broadcasted_iota(jnp.int32, (NGRAPH, BR), 0) == bvals
            ).astype(jnp.float32)
    g_acc[...] += jnp.dot(mask, hv, preferred_element_type=jnp.float32)

    @pl.when(i == NBLK - 1)
    def _head():
        g = g_acc[...]
        z = jnp.dot(g, pw_ref[...], preferred_element_type=jnp.float32) + pb_ref[...]
        z = jnp.maximum(z, 0.0)
        logits = jnp.dot(z, rw_ref[...], preferred_element_type=jnp.float32) + rb_ref[...]
        col = lax.broadcasted_iota(jnp.int32, (NGRAPH, NHID), 1)
        logits = jnp.where(col < NCLASS, logits, jnp.float32(-1e30))
        m = jnp.max(logits, axis=1, keepdims=True)
        lse = m + jnp.log(jnp.sum(jnp.exp(logits - m), axis=1, keepdims=True))
        o_ref[...] = logits - lse


def _tc_mlp_pool_head(p, h, w1, b1, w2, b2, batch3, post_w, post_b,
                      rw_pad, rb_pad):
    return pl.pallas_call(
        _mlp_pool_head_body,
        grid=(NBLK,),
        in_specs=[
            pl.BlockSpec((NCORES, BR, NHID), lambda i: (0, i, 0)),
            pl.BlockSpec((BR, NHID), lambda i: (i, 0)),
        ] + _WSPECS + [
            pl.BlockSpec((1, 1, BR), lambda i: (i, 0, 0)),
        ] + _WSPECS,
        out_specs=pl.BlockSpec((NGRAPH, NHID), lambda i: (0, 0)),
        out_shape=jax.ShapeDtypeStruct((NGRAPH, NHID), jnp.float32),
        scratch_shapes=[pltpu.VMEM((NGRAPH, NHID), jnp.float32)],
    )(p, h, w1, b1, w2, b2, batch3, post_w, post_b, rw_pad, rb_pad)


def kernel(x, edge_index, batch, conv_w1, conv_b1, conv_w2, conv_b2,
           post_w, post_b, read_w, read_b):
    pad_src = jnp.broadcast_to(N + jnp.arange(PADS_W, dtype=jnp.int32),
                               (NWORKERS, PADS_W))
    pad_dst = (jnp.arange(NWORKERS, dtype=jnp.int32)[:, None] * 313
               + jnp.arange(PADS_W, dtype=jnp.int32)[None, :]) % N
    src2 = jnp.concatenate(
        [edge_index[0].reshape(NWORKERS, EDGES_W), pad_src],
        axis=1).reshape(NCHUNKS, CHUNK)
    dst2 = jnp.concatenate(
        [edge_index[1].reshape(NWORKERS, EDGES_W), pad_dst],
        axis=1).reshape(NCHUNKS, CHUNK)
    h = jnp.pad(x, ((0, N_PAD - N), (0, 0)))
    for l in range(NLAYER - 1):
        p = _sc_aggregate(h, src2, dst2)
        h = _tc_mlp(p, h, conv_w1[l], conv_b1[l].reshape(1, NHID),
                    conv_w2[l], conv_b2[l].reshape(1, NHID))

    p = _sc_aggregate(h, src2, dst2)
    batch3 = jnp.pad(batch, (0, N_PAD - N)).reshape(NBLK, 1, BR)
    rw_pad = jnp.zeros((NHID, NHID), jnp.float32).at[:, :NCLASS].set(read_w)
    rb_pad = jnp.zeros((1, NHID), jnp.float32).at[0, :NCLASS].set(read_b)
    out = _tc_mlp_pool_head(
        p, h, conv_w1[NLAYER - 1], conv_b1[NLAYER - 1].reshape(1, NHID),
        conv_w2[NLAYER - 1], conv_b2[NLAYER - 1].reshape(1, NHID),
        batch3, post_w, post_b.reshape(1, NHID), rw_pad, rb_pad)
    return out[:, :NCLASS]


# speedup vs baseline: 3.8730x; 1.0211x over previous
import functools

import jax
import jax.numpy as jnp
from jax import lax
from jax.experimental import pallas as pl
from jax.experimental.pallas import tpu as pltpu
from jax.experimental.pallas import tpu_sc as plsc

N = 10000
E = 320000
NHID = 128
NLAYER = 3
NGRAPH = 128
NCLASS = 10

CHUNK = 128
NCORES = 2
NSUB = 16
NWORKERS = NCORES * NSUB
CHLOC = 10240 // CHUNK
NCHUNKS = NWORKERS * CHLOC
E_PAD = NCHUNKS * CHUNK
NBUF = 2
CHHALF = CHLOC // 2
EDGES_W = E // NWORKERS
PADS_W = CHLOC * CHUNK - EDGES_W
N_PAD = N + PADS_W
ROWS_T = N_PAD // NSUB

BR = 1024
NBLK = N_PAD // BR


def _sc_aggregate(h_pad, src2, dst2):
    mesh = plsc.VectorSubcoreMesh(core_axis_name="c", subcore_axis_name="s")

    @functools.partial(
        pl.kernel,
        mesh=mesh,
        out_type=jax.ShapeDtypeStruct((NCORES, N_PAD, NHID), jnp.float32),
        scratch_types=[
            pltpu.VMEM((CHHALF, CHUNK), jnp.int32),
            pltpu.VMEM((CHHALF, CHUNK), jnp.int32),
        ] + [pltpu.VMEM((CHUNK, NHID), jnp.float32) for _ in range(NBUF)] + [
            pltpu.VMEM_SHARED((N_PAD, NHID), jnp.float32),
        ] + [pltpu.SemaphoreType.DMA for _ in range(NBUF)],
    )
    def agg_kernel(h_hbm, src_hbm, dst_hbm, out_hbm, src_buf, dst_buf,
                   *scratches):
        rows = scratches[:NBUF]
        acc = scratches[NBUF]
        gsem = scratches[NBUF + 1:2 * NBUF + 1]

        c = lax.axis_index("c")
        s = lax.axis_index("s")
        w = s * NCORES + c

        row0 = pl.multiple_of(s * ROWS_T, 8)
        zbuf = rows[0]

        @pl.loop(0, CHUNK)
        def _zr(r):
            @pl.loop(0, NHID, step=16)
            def _zc(cc):
                zbuf[r, pl.ds(cc, 16)] = jnp.zeros((16,), jnp.float32)

        for k in range(ROWS_T // CHUNK):
            pltpu.sync_copy(zbuf, acc.at[pl.ds(row0 + k * CHUNK, CHUNK)])
        plsc.subcore_barrier()

        def g_start(j, b):
            pltpu.async_copy(h_hbm.at[src_buf.at[j]], rows[b], gsem[b])

        def g_wait(b):
            pltpu.make_async_copy(h_hbm.at[src_buf.at[0]], rows[b],
                                  gsem[b]).wait()

        def s_add(j, b):
            pltpu.sync_copy(rows[b], acc.at[dst_buf.at[j]], add=True)

        for half in range(2):
            ch0 = pl.multiple_of(w * CHLOC + half * CHHALF, 8)
            pltpu.sync_copy(src_hbm.at[pl.ds(ch0, CHHALF)], src_buf)
            pltpu.sync_copy(dst_hbm.at[pl.ds(ch0, CHHALF)], dst_buf)

            for b in range(NBUF):
                g_start(b, b)

            @pl.loop(0, CHHALF - NBUF, step=NBUF)
            def _ring(g):
                for b in range(NBUF):
                    j = g + b
                    g_wait(b)
                    s_add(j, b)
                    g_start(j + NBUF, b)

            for b in range(NBUF):
                j = CHHALF - NBUF + b
                g_wait(b)
                s_add(j, b)

        plsc.subcore_barrier()

        pltpu.sync_copy(acc.at[pl.ds(row0, ROWS_T)],
                        out_hbm.at[c, pl.ds(row0, ROWS_T)])

    return agg_kernel(h_pad, src2, dst2)


def _mlp_block(i, p_ref, h_ref, w1_ref, b1_ref, w2_ref, b2_ref):
    z = p_ref[0] + p_ref[1] + h_ref[...]
    z = jnp.dot(z, w1_ref[...], preferred_element_type=jnp.float32) + b1_ref[...]
    z = jnp.maximum(z, 0.0)
    z = jnp.dot(z, w2_ref[...], preferred_element_type=jnp.float32) + b2_ref[...]
    z = jnp.maximum(z, 0.0)
    row = i * BR + lax.broadcasted_iota(jnp.int32, (BR, NHID), 0)
    return jnp.where(row < N, z, 0.0)


def _mlp_body(p_ref, h_ref, w1_ref, b1_ref, w2_ref, b2_ref, o_ref):
    o_ref[...] = _mlp_block(pl.program_id(0), p_ref, h_ref,
                            w1_ref, b1_ref, w2_ref, b2_ref)


_WSPECS = [
    pl.BlockSpec((NHID, NHID), lambda i: (0, 0)),
    pl.BlockSpec((1, NHID), lambda i: (0, 0)),
    pl.BlockSpec((NHID, NHID), lambda i: (0, 0)),
    pl.BlockSpec((1, NHID), lambda i: (0, 0)),
]


def _tc_mlp(p, h, w1, b1, w2, b2):
    return pl.pallas_call(
        _mlp_body,
        grid=(NBLK,),
        in_specs=[
            pl.BlockSpec((NCORES, BR, NHID), lambda i: (0, i, 0)),
            pl.BlockSpec((BR, NHID), lambda i: (i, 0)),
        ] + _WSPECS,
        out_specs=pl.BlockSpec((BR, NHID), lambda i: (i, 0)),
        out_shape=jax.ShapeDtypeStruct((N_PAD, NHID), jnp.float32),
    )(p, h, w1, b1, w2, b2)


def _mlp_pool_head_body(p_ref, h_ref, w1_ref, b1_ref, w2_ref, b2_ref,
                        b_ref, pw_ref, pb_ref, rw_ref, rb_ref, o_ref, g_acc):
    i = pl.program_id(0)

    @pl.when(i == 0)
    def _init():
        g_acc[...] = jnp.zeros_like(g_acc)

    hv = _mlp_block(i, p_ref, h_ref, w1_ref, b1_ref, w2_ref, b2_ref)
    bvals = b_ref[0]
    mask = (lax.broadcasted_iota(jnp.int32, (NGRAPH, BR), 0) == bvals
            ).astype(jnp.float32)
    g_acc[...] += jnp.dot(mask, hv, preferred_element_type=jnp.float32)

    @pl.when(i == NBLK - 1)
    def _head():
        g = g_acc[...]
        z = jnp.dot(g, pw_ref[...], preferred_element_type=jnp.float32) + pb_ref[...]
        z = jnp.maximum(z, 0.0)
        logits = jnp.dot(z, rw_ref[...], preferred_element_type=jnp.float32) + rb_ref[...]
        col = lax.broadcasted_iota(jnp.int32, (NGRAPH, NHID), 1)
        logits = jnp.where(col < NCLASS, logits, jnp.float32(-1e30))
        m = jnp.max(logits, axis=1, keepdims=True)
        lse = m + jnp.log(jnp.sum(jnp.exp(logits - m), axis=1, keepdims=True))
        o_ref[...] = logits - lse


def _tc_mlp_pool_head(p, h, w1, b1, w2, b2, batch3, post_w, post_b,
                      rw_pad, rb_pad):
    return pl.pallas_call(
        _mlp_pool_head_body,
        grid=(NBLK,),
        in_specs=[
            pl.BlockSpec((NCORES, BR, NHID), lambda i: (0, i, 0)),
            pl.BlockSpec((BR, NHID), lambda i: (i, 0)),
        ] + _WSPECS + [
            pl.BlockSpec((1, 1, BR), lambda i: (i, 0, 0)),
        ] + _WSPECS,
        out_specs=pl.BlockSpec((NGRAPH, NHID), lambda i: (0, 0)),
        out_shape=jax.ShapeDtypeStruct((NGRAPH, NHID), jnp.float32),
        scratch_shapes=[pltpu.VMEM((NGRAPH, NHID), jnp.float32)],
    )(p, h, w1, b1, w2, b2, batch3, post_w, post_b, rw_pad, rb_pad)


def kernel(x, edge_index, batch, conv_w1, conv_b1, conv_w2, conv_b2,
           post_w, post_b, read_w, read_b):
    pad_src = jnp.broadcast_to(N + jnp.arange(PADS_W, dtype=jnp.int32),
                               (NWORKERS, PADS_W))
    pad_dst = (jnp.arange(NWORKERS, dtype=jnp.int32)[:, None] * 313
               + jnp.arange(PADS_W, dtype=jnp.int32)[None, :]) % N
    src2 = jnp.concatenate(
        [edge_index[0].reshape(NWORKERS, EDGES_W), pad_src],
        axis=1).reshape(NCHUNKS, CHUNK)
    dst2 = jnp.concatenate(
        [edge_index[1].reshape(NWORKERS, EDGES_W), pad_dst],
        axis=1).reshape(NCHUNKS, CHUNK)
    h = jnp.pad(x, ((0, N_PAD - N), (0, 0)))
    for l in range(NLAYER - 1):
        p = _sc_aggregate(h, src2, dst2)
        h = _tc_mlp(p, h, conv_w1[l], conv_b1[l].reshape(1, NHID),
                    conv_w2[l], conv_b2[l].reshape(1, NHID))

    p = _sc_aggregate(h, src2, dst2)
    batch3 = jnp.pad(batch, (0, N_PAD - N)).reshape(NBLK, 1, BR)
    rw_pad = jnp.zeros((NHID, NHID), jnp.float32).at[:, :NCLASS].set(read_w)
    rb_pad = jnp.zeros((1, NHID), jnp.float32).at[0, :NCLASS].set(read_b)
    out = _tc_mlp_pool_head(
        p, h, conv_w1[NLAYER - 1], conv_b1[NLAYER - 1].reshape(1, NHID),
        conv_w2[NLAYER - 1], conv_b2[NLAYER - 1].reshape(1, NHID),
        batch3, post_w, post_b.reshape(1, NHID), rw_pad, rb_pad)
    return out[:, :NCLASS]
